# baseline (speedup 1.0000x reference)
import functools

import jax
import jax.numpy as jnp
from jax.experimental import pallas as pl
from jax.experimental.pallas import tpu as pltpu

N_EXP = 64
D_MODEL = 1024
D_FF = 1024
B_ROWS = 128
NB = 4096 // B_ROWS + (N_EXP - 1) + 1
NP_ROWS = (NB + 1) * B_ROWS


def _ffn_body(be_ref, br_ref, xs_ref, w1_ref, w2_ref, w3_ref, out_ref):
    del be_ref, br_ref
    xb = xs_ref[0]
    g = jax.nn.silu(jnp.dot(xb, w1_ref[0], preferred_element_type=jnp.float32))
    v = jnp.dot(xb, w2_ref[0], preferred_element_type=jnp.float32)
    out_ref[0] = jnp.dot(g * v, w3_ref[0], preferred_element_type=jnp.float32)


@jax.jit
def _grouped_ffn(block_expert, block_row, xs, w1, w2, w3):
    grid_spec = pltpu.PrefetchScalarGridSpec(
        num_scalar_prefetch=2,
        grid=(NB,),
        in_specs=[
            pl.BlockSpec((1, B_ROWS, D_MODEL), lambda b, be, br: (br[b], 0, 0)),
            pl.BlockSpec((1, D_MODEL, D_FF), lambda b, be, br: (be[b], 0, 0)),
            pl.BlockSpec((1, D_MODEL, D_FF), lambda b, be, br: (be[b], 0, 0)),
            pl.BlockSpec((1, D_FF, D_MODEL), lambda b, be, br: (be[b], 0, 0)),
        ],
        out_specs=pl.BlockSpec(
            (1, B_ROWS, D_MODEL), lambda b, be, br: (br[b], 0, 0)),
    )
    return pl.pallas_call(
        _ffn_body,
        grid_spec=grid_spec,
        out_shape=jax.ShapeDtypeStruct((NB + 1, B_ROWS, D_MODEL), jnp.float32),
        compiler_params=pltpu.CompilerParams(
            dimension_semantics=("arbitrary",)),
    )(block_expert, block_row, xs, w1, w2, w3)


def kernel(x, expert_indices, expert_weights, w1, w2, w3):
    n_tokens, d_model = x.shape
    top_k = expert_indices.shape[1]
    na = n_tokens * top_k

    flat_e = expert_indices.reshape(-1).astype(jnp.int32)
    flat_w = expert_weights.reshape(-1)

    onehot = (flat_e[None, :] == jnp.arange(N_EXP, dtype=jnp.int32)[:, None])
    csum = jnp.cumsum(onehot.astype(jnp.int32), axis=1)
    rank = jnp.take_along_axis(csum, flat_e[None, :], axis=0)[0] - 1
    counts = csum[:, -1]

    nblk = (counts + B_ROWS - 1) // B_ROWS
    pcnt = nblk * B_ROWS
    pstart = jnp.concatenate(
        [jnp.zeros((1,), jnp.int32), jnp.cumsum(pcnt)[:-1].astype(jnp.int32)])
    bstart = pstart // B_ROWS

    pos = pstart[flat_e] + rank

    nb_used = bstart[-1] + nblk[-1]
    gb = jnp.arange(NB, dtype=jnp.int32)
    own = jnp.searchsorted(bstart, gb, side="right").astype(jnp.int32) - 1
    valid = gb < nb_used
    last_e = jnp.argmax(jnp.where(counts > 0, jnp.arange(N_EXP), -1)).astype(
        jnp.int32)
    block_expert = jnp.where(valid, own, last_e)
    block_row = jnp.where(valid, gb, NB).astype(jnp.int32)

    pos2 = pos.reshape(n_tokens, top_k)
    xs = jnp.zeros((NP_ROWS, d_model), x.dtype)
    for k in range(top_k):
        xs = xs.at[pos2[:, k]].set(x)
    xs = xs.reshape(NB + 1, B_ROWS, d_model)

    ys = _grouped_ffn(block_expert, block_row, xs, w1, w2, w3)
    ys = ys.reshape(NP_ROWS, d_model)

    out = (ys[pos] * flat_w[:, None]).reshape(n_tokens, top_k, d_model).sum(
        axis=1)
    return out


# speedup vs baseline: 1.3991x; 1.3991x over previous
import functools

import jax
import jax.numpy as jnp
from jax import lax
from jax.experimental import pallas as pl
from jax.experimental.pallas import tpu as pltpu
from jax.experimental.pallas import tpu_sc as plsc

N_EXP = 64
D_MODEL = 1024
D_FF = 1024
B_ROWS = 128
NB = 4096 // B_ROWS + (N_EXP - 1) + 1
NP_ROWS = (NB + 1) * B_ROWS
NA = 4096
NT = 2048

A_TILES = 16
A_PER = NA // A_TILES
GCH = 32
N_GCH = A_PER // GCH


def _route_body(ei_hbm, x_hbm, xs_hbm, pos_hbm, be_hbm, br_hbm,
                keys_v, rank_v, cnt_v, hist_sh, hist_v, base_v, nblk_v,
                pos_v, posr_v, tokr_v, rows_v, be_v, br_v, sem):
    wid = lax.axis_index("s")
    a0 = wid * A_PER

    pltpu.sync_copy(ei_hbm.at[pl.ds(a0, A_PER)], keys_v)

    zeros16 = jnp.zeros((16,), jnp.int32)
    lane = lax.iota(jnp.int32, 16)
    lane0 = lane == 0
    for j in range(N_EXP // 16):
        cnt_v[pl.ds(16 * j, 16)] = zeros16

    def rank_step(i, carry):
        iv = jnp.full((16,), i, jnp.int32)
        e = plsc.load_gather(keys_v, [iv])
        c = plsc.load_gather(cnt_v, [e])
        plsc.store_scatter(rank_v, [iv], c, mask=lane0)
        plsc.store_scatter(cnt_v, [e], c + 1, mask=lane0)
        return carry

    lax.fori_loop(0, A_PER, rank_step, 0)

    pltpu.sync_copy(cnt_v, hist_sh.at[pl.ds(wid * N_EXP, N_EXP)])
    plsc.subcore_barrier()
    pltpu.sync_copy(hist_sh, hist_v)

    widv = jnp.full((16,), wid, jnp.int32)
    carry_s = 0
    for j in range(N_EXP // 16):
        tot = jnp.zeros((16,), jnp.int32)
        pre = jnp.zeros((16,), jnp.int32)
        for t in range(A_TILES):
            h = hist_v[pl.ds(t * N_EXP + 16 * j, 16)]
            pre = pre + jnp.where(jnp.full((16,), t, jnp.int32) < widv, h,
                                  zeros16)
            tot = tot + h
        nblk = (tot + (B_ROWS - 1)) // B_ROWS
        inc = plsc.cumsum(nblk)
        excl_blocks = inc - nblk + carry_s
        base_v[pl.ds(16 * j, 16)] = excl_blocks * B_ROWS + pre
        nblk_v[pl.ds(16 * j, 16)] = nblk
        carry_s = carry_s + inc[15]

    for j in range(A_PER // 16):
        kv = keys_v[pl.ds(16 * j, 16)]
        b = plsc.load_gather(base_v, [kv])
        pv = b + rank_v[pl.ds(16 * j, 16)]
        pos_v[pl.ds(16 * j, 16)] = pv
        posr_v[j // 2, pl.ds(16 * (j % 2), 16)] = pv
        g = jnp.full((16,), a0 + 16 * j, jnp.int32) + lane
        tokr_v[j // 2, pl.ds(16 * (j % 2), 16)] = lax.rem(
            g, jnp.full((16,), NT, jnp.int32))

    pltpu.sync_copy(pos_v, pos_hbm.at[pl.ds(a0, A_PER)])

    for c in range(N_GCH):
        pltpu.async_copy(x_hbm.at[tokr_v.at[c]], rows_v, sem).wait()
        pltpu.async_copy(rows_v, xs_hbm.at[posr_v.at[c]], sem).wait()

    @pl.when(wid == 0)
    def _():
        def expert_step(e, carry):
            cur, last = carry
            ev = jnp.full((16,), e, jnp.int32)
            nb = plsc.load_gather(nblk_v, [ev])[0]

            def blk_step(k, _):
                plsc.store_scatter(
                    be_v, [jnp.full((16,), cur + k, jnp.int32)], ev,
                    mask=lane0)
                return 0

            lax.fori_loop(0, nb, blk_step, 0)
            return cur + nb, jnp.where(nb > 0, e, last)

        cur, last = lax.fori_loop(0, N_EXP, expert_step, (0, 0))
        lastv = jnp.full((16,), last, jnp.int32)

        def fill_step(b, _):
            plsc.store_scatter(
                be_v, [jnp.full((16,), b, jnp.int32)], lastv, mask=lane0)
            return 0

        lax.fori_loop(cur, NB, fill_step, 0)

        curv = jnp.full((16,), cur, jnp.int32)
        for j in range(NB // 16):
            bv = jnp.full((16,), 16 * j, jnp.int32) + lane
            br_v[pl.ds(16 * j, 16)] = jnp.where(
                bv < curv, bv, jnp.full((16,), NB, jnp.int32))
        pltpu.sync_copy(be_v, be_hbm)
        pltpu.sync_copy(br_v, br_hbm)


@functools.cache
def _make_route():
    mesh = plsc.VectorSubcoreMesh(
        core_axis_name="c", subcore_axis_name="s",
        num_cores=1, num_subcores=A_TILES)
    return functools.partial(
        pl.kernel,
        mesh=mesh,
        out_type=(
            jax.ShapeDtypeStruct((NP_ROWS, D_MODEL), jnp.float32),
            jax.ShapeDtypeStruct((NA,), jnp.int32),
            jax.ShapeDtypeStruct((NB,), jnp.int32),
            jax.ShapeDtypeStruct((NB,), jnp.int32),
        ),
        scratch_types=[
            pltpu.VMEM((A_PER,), jnp.int32),
            pltpu.VMEM((A_PER,), jnp.int32),
            pltpu.VMEM((N_EXP,), jnp.int32),
            pltpu.VMEM_SHARED((A_TILES * N_EXP,), jnp.int32),
            pltpu.VMEM((A_TILES * N_EXP,), jnp.int32),
            pltpu.VMEM((N_EXP,), jnp.int32),
            pltpu.VMEM((N_EXP,), jnp.int32),
            pltpu.VMEM((A_PER,), jnp.int32),
            pltpu.VMEM((N_GCH, GCH), jnp.int32),
            pltpu.VMEM((N_GCH, GCH), jnp.int32),
            pltpu.VMEM((GCH, D_MODEL), jnp.float32),
            pltpu.VMEM((NB,), jnp.int32),
            pltpu.VMEM((NB,), jnp.int32),
            pltpu.SemaphoreType.DMA,
        ],
        compiler_params=pltpu.CompilerParams(needs_layout_passes=False),
    )(_route_body)


def _ffn_body(be_ref, br_ref, xs_ref, w1_ref, w2_ref, w3_ref, out_ref):
    del be_ref, br_ref
    xb = xs_ref[0]
    g = jax.nn.silu(jnp.dot(xb, w1_ref[0], preferred_element_type=jnp.float32))
    v = jnp.dot(xb, w2_ref[0], preferred_element_type=jnp.float32)
    out_ref[0] = jnp.dot(g * v, w3_ref[0], preferred_element_type=jnp.float32)


@jax.jit
def _grouped_ffn(block_expert, block_row, xs, w1, w2, w3):
    grid_spec = pltpu.PrefetchScalarGridSpec(
        num_scalar_prefetch=2,
        grid=(NB,),
        in_specs=[
            pl.BlockSpec((1, B_ROWS, D_MODEL), lambda b, be, br: (br[b], 0, 0)),
            pl.BlockSpec((1, D_MODEL, D_FF), lambda b, be, br: (be[b], 0, 0)),
            pl.BlockSpec((1, D_MODEL, D_FF), lambda b, be, br: (be[b], 0, 0)),
            pl.BlockSpec((1, D_FF, D_MODEL), lambda b, be, br: (be[b], 0, 0)),
        ],
        out_specs=pl.BlockSpec(
            (1, B_ROWS, D_MODEL), lambda b, be, br: (br[b], 0, 0)),
    )
    return pl.pallas_call(
        _ffn_body,
        grid_spec=grid_spec,
        out_shape=jax.ShapeDtypeStruct((NB + 1, B_ROWS, D_MODEL), jnp.float32),
        compiler_params=pltpu.CompilerParams(
            dimension_semantics=("arbitrary",)),
    )(block_expert, block_row, xs, w1, w2, w3)


C_TILES = 32
C_PER = NA // C_TILES
C_GCH = 32


def _ungroup_body(ys_hbm, pos_hbm, yu_hbm, pos_v, rows_v, sem):
    wid = lax.axis_index("s") * 2 + lax.axis_index("c")
    a0 = wid * C_PER
    pltpu.sync_copy(pos_hbm.at[pl.ds(a0, C_PER)], pos_v)
    for c in range(C_PER // C_GCH):
        pltpu.async_copy(ys_hbm.at[pos_v.at[pl.ds(c * C_GCH, C_GCH)]],
                         rows_v, sem).wait()
        pltpu.sync_copy(rows_v, yu_hbm.at[pl.ds(a0 + c * C_GCH, C_GCH)])


@functools.cache
def _make_ungroup():
    mesh = plsc.VectorSubcoreMesh(
        core_axis_name="c", subcore_axis_name="s",
        num_cores=2, num_subcores=16)
    return functools.partial(
        pl.kernel,
        mesh=mesh,
        out_type=jax.ShapeDtypeStruct((NA, D_MODEL), jnp.float32),
        scratch_types=[
            pltpu.VMEM((C_PER,), jnp.int32),
            pltpu.VMEM((C_GCH, D_MODEL), jnp.float32),
            pltpu.SemaphoreType.DMA,
        ],
        compiler_params=pltpu.CompilerParams(needs_layout_passes=False),
    )(_ungroup_body)


def _combine_body(ya_ref, yb_ref, wa_ref, wb_ref, out_ref):
    out_ref[...] = ya_ref[...] * wa_ref[...] + yb_ref[...] * wb_ref[...]


@jax.jit
def _combine(yu, wa, wb):
    nblk = NT // B_ROWS
    return pl.pallas_call(
        _combine_body,
        grid=(nblk,),
        in_specs=[
            pl.BlockSpec((B_ROWS, D_MODEL), lambda b: (b, 0)),
            pl.BlockSpec((B_ROWS, D_MODEL), lambda b: (b + nblk, 0)),
            pl.BlockSpec((B_ROWS, 1), lambda b: (b, 0)),
            pl.BlockSpec((B_ROWS, 1), lambda b: (b, 0)),
        ],
        out_specs=pl.BlockSpec((B_ROWS, D_MODEL), lambda b: (b, 0)),
        out_shape=jax.ShapeDtypeStruct((NT, D_MODEL), jnp.float32),
        compiler_params=pltpu.CompilerParams(
            dimension_semantics=("arbitrary",)),
    )(yu, yu, wa, wb)


def kernel(x, expert_indices, expert_weights, w1, w2, w3):
    n_tokens, d_model = x.shape
    ei_flat = expert_indices.T.reshape(-1).astype(jnp.int32)

    xs, pos, block_expert, block_row = _make_route()(ei_flat, x)
    xs = xs.reshape(NB + 1, B_ROWS, d_model)

    ys = _grouped_ffn(block_expert, block_row, xs, w1, w2, w3)
    ys = ys.reshape(NP_ROWS, d_model)

    yu = _make_ungroup()(ys, pos)

    wa = expert_weights[:, 0:1]
    wb = expert_weights[:, 1:2]
    return _combine(yu, wa, wb)


# speedup vs baseline: 1.4215x; 1.0160x over previous
import functools

import jax
import jax.numpy as jnp
from jax import lax
from jax.experimental import pallas as pl
from jax.experimental.pallas import tpu as pltpu
from jax.experimental.pallas import tpu_sc as plsc

N_EXP = 64
D_MODEL = 1024
D_FF = 1024
B_ROWS = 128
NB = 4096 // B_ROWS + (N_EXP - 1) + 1
NP_ROWS = (NB + 1) * B_ROWS
NA = 4096
NT = 2048

A_TILES = 16
A_PER = NA // A_TILES
GCH = 32
N_GCH = A_PER // GCH


def _route_body(ei_hbm, x_hbm, xs_hbm, pos_hbm, be_hbm, br_hbm,
                keys_v, rank_v, cnt_v, hist_sh, hist_v, base_v, nblk_v,
                pos_v, posr_v, tokr_v, rows_v, rows2_v, be_v, br_v, sem,
                sem2):
    wid = lax.axis_index("s")
    a0 = wid * A_PER

    pltpu.sync_copy(ei_hbm.at[pl.ds(a0, A_PER)], keys_v)

    zeros16 = jnp.zeros((16,), jnp.int32)
    lane = lax.iota(jnp.int32, 16)
    lane0 = lane == 0
    for j in range(N_EXP // 16):
        cnt_v[pl.ds(16 * j, 16)] = zeros16

    def rank_step(i, carry):
        iv = jnp.full((16,), i, jnp.int32)
        e = plsc.load_gather(keys_v, [iv])
        c = plsc.load_gather(cnt_v, [e])
        plsc.store_scatter(rank_v, [iv], c, mask=lane0)
        plsc.store_scatter(cnt_v, [e], c + 1, mask=lane0)
        return carry

    lax.fori_loop(0, A_PER, rank_step, 0)

    pltpu.sync_copy(cnt_v, hist_sh.at[pl.ds(wid * N_EXP, N_EXP)])
    plsc.subcore_barrier()
    pltpu.sync_copy(hist_sh, hist_v)

    widv = jnp.full((16,), wid, jnp.int32)
    carry_s = 0
    for j in range(N_EXP // 16):
        tot = jnp.zeros((16,), jnp.int32)
        pre = jnp.zeros((16,), jnp.int32)
        for t in range(A_TILES):
            h = hist_v[pl.ds(t * N_EXP + 16 * j, 16)]
            pre = pre + jnp.where(jnp.full((16,), t, jnp.int32) < widv, h,
                                  zeros16)
            tot = tot + h
        nblk = (tot + (B_ROWS - 1)) // B_ROWS
        inc = plsc.cumsum(nblk)
        excl_blocks = inc - nblk + carry_s
        base_v[pl.ds(16 * j, 16)] = excl_blocks * B_ROWS + pre
        nblk_v[pl.ds(16 * j, 16)] = nblk
        carry_s = carry_s + inc[15]

    for j in range(A_PER // 16):
        kv = keys_v[pl.ds(16 * j, 16)]
        b = plsc.load_gather(base_v, [kv])
        pv = b + rank_v[pl.ds(16 * j, 16)]
        pos_v[pl.ds(16 * j, 16)] = pv
        posr_v[j // 2, pl.ds(16 * (j % 2), 16)] = pv
        g = jnp.full((16,), a0 + 16 * j, jnp.int32) + lane
        tokr_v[j // 2, pl.ds(16 * (j % 2), 16)] = lax.rem(
            g, jnp.full((16,), NT, jnp.int32))

    pltpu.sync_copy(pos_v, pos_hbm.at[pl.ds(a0, A_PER)])

    bufs = (rows_v, rows2_v)
    hg = [None] * N_GCH
    hs = [None] * N_GCH
    hg[0] = pltpu.async_copy(x_hbm.at[tokr_v.at[0]], bufs[0], sem)
    for c in range(N_GCH):
        cur = bufs[c % 2]
        hg[c].wait()
        hs[c] = pltpu.async_copy(cur, xs_hbm.at[posr_v.at[c]], sem2)
        if c + 1 < N_GCH:
            if c >= 1:
                hs[c - 1].wait()
            hg[c + 1] = pltpu.async_copy(
                x_hbm.at[tokr_v.at[c + 1]], bufs[(c + 1) % 2], sem)
    hs[N_GCH - 2].wait()
    hs[N_GCH - 1].wait()

    @pl.when(wid == 0)
    def _():
        def expert_step(e, carry):
            cur, last = carry
            ev = jnp.full((16,), e, jnp.int32)
            nb = plsc.load_gather(nblk_v, [ev])[0]

            def blk_step(k, _):
                plsc.store_scatter(
                    be_v, [jnp.full((16,), cur + k, jnp.int32)], ev,
                    mask=lane0)
                return 0

            lax.fori_loop(0, nb, blk_step, 0)
            return cur + nb, jnp.where(nb > 0, e, last)

        cur, last = lax.fori_loop(0, N_EXP, expert_step, (0, 0))
        lastv = jnp.full((16,), last, jnp.int32)

        def fill_step(b, _):
            plsc.store_scatter(
                be_v, [jnp.full((16,), b, jnp.int32)], lastv, mask=lane0)
            return 0

        lax.fori_loop(cur, NB, fill_step, 0)

        curv = jnp.full((16,), cur, jnp.int32)
        for j in range(NB // 16):
            bv = jnp.full((16,), 16 * j, jnp.int32) + lane
            br_v[pl.ds(16 * j, 16)] = jnp.where(
                bv < curv, bv, jnp.full((16,), NB, jnp.int32))
        pltpu.sync_copy(be_v, be_hbm)
        pltpu.sync_copy(br_v, br_hbm)


@functools.cache
def _make_route():
    mesh = plsc.VectorSubcoreMesh(
        core_axis_name="c", subcore_axis_name="s",
        num_cores=1, num_subcores=A_TILES)
    return functools.partial(
        pl.kernel,
        mesh=mesh,
        out_type=(
            jax.ShapeDtypeStruct((NP_ROWS, D_MODEL), jnp.float32),
            jax.ShapeDtypeStruct((NA,), jnp.int32),
            jax.ShapeDtypeStruct((NB,), jnp.int32),
            jax.ShapeDtypeStruct((NB,), jnp.int32),
        ),
        scratch_types=[
            pltpu.VMEM((A_PER,), jnp.int32),
            pltpu.VMEM((A_PER,), jnp.int32),
            pltpu.VMEM((N_EXP,), jnp.int32),
            pltpu.VMEM_SHARED((A_TILES * N_EXP,), jnp.int32),
            pltpu.VMEM((A_TILES * N_EXP,), jnp.int32),
            pltpu.VMEM((N_EXP,), jnp.int32),
            pltpu.VMEM((N_EXP,), jnp.int32),
            pltpu.VMEM((A_PER,), jnp.int32),
            pltpu.VMEM((N_GCH, GCH), jnp.int32),
            pltpu.VMEM((N_GCH, GCH), jnp.int32),
            pltpu.VMEM((GCH, D_MODEL), jnp.float32),
            pltpu.VMEM((GCH, D_MODEL), jnp.float32),
            pltpu.VMEM((NB,), jnp.int32),
            pltpu.VMEM((NB,), jnp.int32),
            pltpu.SemaphoreType.DMA,
            pltpu.SemaphoreType.DMA,
        ],
        compiler_params=pltpu.CompilerParams(needs_layout_passes=False),
    )(_route_body)


def _ffn_body(be_ref, br_ref, xs_ref, w1_ref, w2_ref, w3_ref, out_ref):
    del be_ref, br_ref
    xb = xs_ref[0]
    g = jax.nn.silu(jnp.dot(xb, w1_ref[0], preferred_element_type=jnp.float32))
    v = jnp.dot(xb, w2_ref[0], preferred_element_type=jnp.float32)
    out_ref[0] = jnp.dot(g * v, w3_ref[0], preferred_element_type=jnp.float32)


@jax.jit
def _grouped_ffn(block_expert, block_row, xs, w1, w2, w3):
    grid_spec = pltpu.PrefetchScalarGridSpec(
        num_scalar_prefetch=2,
        grid=(NB,),
        in_specs=[
            pl.BlockSpec((1, B_ROWS, D_MODEL), lambda b, be, br: (br[b], 0, 0)),
            pl.BlockSpec((1, D_MODEL, D_FF), lambda b, be, br: (be[b], 0, 0)),
            pl.BlockSpec((1, D_MODEL, D_FF), lambda b, be, br: (be[b], 0, 0)),
            pl.BlockSpec((1, D_FF, D_MODEL), lambda b, be, br: (be[b], 0, 0)),
        ],
        out_specs=pl.BlockSpec(
            (1, B_ROWS, D_MODEL), lambda b, be, br: (br[b], 0, 0)),
    )
    return pl.pallas_call(
        _ffn_body,
        grid_spec=grid_spec,
        out_shape=jax.ShapeDtypeStruct((NB + 1, B_ROWS, D_MODEL), jnp.float32),
        compiler_params=pltpu.CompilerParams(
            dimension_semantics=("arbitrary",)),
    )(block_expert, block_row, xs, w1, w2, w3)


C_TILES = 32
C_PER = NA // C_TILES
C_GCH = 32


def _ungroup_body(ys_hbm, pos_hbm, yu_hbm, pos_v, rows_v, rows2_v, sem,
                  sem2):
    wid = lax.axis_index("s") * 2 + lax.axis_index("c")
    a0 = wid * C_PER
    pltpu.sync_copy(pos_hbm.at[pl.ds(a0, C_PER)], pos_v)
    n = C_PER // C_GCH
    bufs = (rows_v, rows2_v)
    hg = [None] * n
    hs = [None] * n
    hg[0] = pltpu.async_copy(ys_hbm.at[pos_v.at[pl.ds(0, C_GCH)]], bufs[0],
                             sem)
    for c in range(n):
        cur = bufs[c % 2]
        hg[c].wait()
        hs[c] = pltpu.async_copy(
            cur, yu_hbm.at[pl.ds(a0 + c * C_GCH, C_GCH)], sem2)
        if c + 1 < n:
            if c >= 1:
                hs[c - 1].wait()
            hg[c + 1] = pltpu.async_copy(
                ys_hbm.at[pos_v.at[pl.ds((c + 1) * C_GCH, C_GCH)]],
                bufs[(c + 1) % 2], sem)
    hs[n - 2].wait()
    hs[n - 1].wait()


@functools.cache
def _make_ungroup():
    mesh = plsc.VectorSubcoreMesh(
        core_axis_name="c", subcore_axis_name="s",
        num_cores=2, num_subcores=16)
    return functools.partial(
        pl.kernel,
        mesh=mesh,
        out_type=jax.ShapeDtypeStruct((NA, D_MODEL), jnp.float32),
        scratch_types=[
            pltpu.VMEM((C_PER,), jnp.int32),
            pltpu.VMEM((C_GCH, D_MODEL), jnp.float32),
            pltpu.VMEM((C_GCH, D_MODEL), jnp.float32),
            pltpu.SemaphoreType.DMA,
            pltpu.SemaphoreType.DMA,
        ],
        compiler_params=pltpu.CompilerParams(needs_layout_passes=False),
    )(_ungroup_body)


def _combine_body(ya_ref, yb_ref, wa_ref, wb_ref, out_ref):
    out_ref[...] = ya_ref[...] * wa_ref[...] + yb_ref[...] * wb_ref[...]


@jax.jit
def _combine(yu, wa, wb):
    nblk = NT // B_ROWS
    return pl.pallas_call(
        _combine_body,
        grid=(nblk,),
        in_specs=[
            pl.BlockSpec((B_ROWS, D_MODEL), lambda b: (b, 0)),
            pl.BlockSpec((B_ROWS, D_MODEL), lambda b: (b + nblk, 0)),
            pl.BlockSpec((B_ROWS, 1), lambda b: (b, 0)),
            pl.BlockSpec((B_ROWS, 1), lambda b: (b, 0)),
        ],
        out_specs=pl.BlockSpec((B_ROWS, D_MODEL), lambda b: (b, 0)),
        out_shape=jax.ShapeDtypeStruct((NT, D_MODEL), jnp.float32),
        compiler_params=pltpu.CompilerParams(
            dimension_semantics=("arbitrary",)),
    )(yu, yu, wa, wb)


def kernel(x, expert_indices, expert_weights, w1, w2, w3):
    n_tokens, d_model = x.shape
    ei_flat = expert_indices.T.reshape(-1).astype(jnp.int32)

    xs, pos, block_expert, block_row = _make_route()(ei_flat, x)
    xs = xs.reshape(NB + 1, B_ROWS, d_model)

    ys = _grouped_ffn(block_expert, block_row, xs, w1, w2, w3)
    ys = ys.reshape(NP_ROWS, d_model)

    yu = _make_ungroup()(ys, pos)

    wa = expert_weights[:, 0:1]
    wb = expert_weights[:, 1:2]
    return _combine(yu, wa, wb)


# speedup vs baseline: 1.4467x; 1.0177x over previous
import functools

import jax
import jax.numpy as jnp
from jax import lax
from jax.experimental import pallas as pl
from jax.experimental.pallas import tpu as pltpu
from jax.experimental.pallas import tpu_sc as plsc

N_EXP = 64
D_MODEL = 1024
D_FF = 1024
B_ROWS = 128
NB = 4096 // B_ROWS + (N_EXP - 1) + 1
NP_ROWS = (NB + 1) * B_ROWS
NA = 4096
NT = 2048

A_TILES = 16
A_PER = NA // A_TILES
GCH = 32
N_GCH = A_PER // GCH


def _route_body(ei_hbm, x_hbm, xs_hbm, pos_hbm, be_hbm, br_hbm,
                keys_v, rank_v, cnt_v, hist_sh, hist_v, base_v, nblk_v,
                pos_v, posr_v, tokr_v, rows_v, rows2_v, be_v, br_v, sem,
                sem2):
    wid = lax.axis_index("s")
    a0 = wid * A_PER

    pltpu.sync_copy(ei_hbm.at[pl.ds(a0, A_PER)], keys_v)

    zeros16 = jnp.zeros((16,), jnp.int32)
    lane = lax.iota(jnp.int32, 16)
    lane0 = lane == 0
    for j in range(N_EXP // 16):
        cnt_v[pl.ds(16 * j, 16)] = zeros16

    def rank_step(i, carry):
        iv = jnp.full((16,), i, jnp.int32)
        e = plsc.load_gather(keys_v, [iv])
        c = plsc.load_gather(cnt_v, [e])
        plsc.store_scatter(rank_v, [iv], c, mask=lane0)
        plsc.store_scatter(cnt_v, [e], c + 1, mask=lane0)
        return carry

    lax.fori_loop(0, A_PER, rank_step, 0)

    pltpu.sync_copy(cnt_v, hist_sh.at[pl.ds(wid * N_EXP, N_EXP)])
    plsc.subcore_barrier()
    pltpu.sync_copy(hist_sh, hist_v)

    widv = jnp.full((16,), wid, jnp.int32)
    carry_s = 0
    for j in range(N_EXP // 16):
        tot = jnp.zeros((16,), jnp.int32)
        pre = jnp.zeros((16,), jnp.int32)
        for t in range(A_TILES):
            h = hist_v[pl.ds(t * N_EXP + 16 * j, 16)]
            pre = pre + jnp.where(jnp.full((16,), t, jnp.int32) < widv, h,
                                  zeros16)
            tot = tot + h
        nblk = (tot + (B_ROWS - 1)) // B_ROWS
        inc = plsc.cumsum(nblk)
        excl_blocks = inc - nblk + carry_s
        base_v[pl.ds(16 * j, 16)] = excl_blocks * B_ROWS + pre
        nblk_v[pl.ds(16 * j, 16)] = nblk
        carry_s = carry_s + inc[15]

    for j in range(A_PER // 16):
        kv = keys_v[pl.ds(16 * j, 16)]
        b = plsc.load_gather(base_v, [kv])
        pv = b + rank_v[pl.ds(16 * j, 16)]
        pos_v[pl.ds(16 * j, 16)] = pv
        posr_v[j // 2, pl.ds(16 * (j % 2), 16)] = pv
        g = jnp.full((16,), a0 + 16 * j, jnp.int32) + lane
        tokr_v[j // 2, pl.ds(16 * (j % 2), 16)] = lax.rem(
            g, jnp.full((16,), NT, jnp.int32))

    pltpu.sync_copy(pos_v, pos_hbm.at[pl.ds(a0, A_PER)])

    bufs = (rows_v, rows2_v)
    hg = [None] * N_GCH
    hs = [None] * N_GCH
    hg[0] = pltpu.async_copy(x_hbm.at[tokr_v.at[0]], bufs[0], sem)
    for c in range(N_GCH):
        cur = bufs[c % 2]
        hg[c].wait()
        hs[c] = pltpu.async_copy(cur, xs_hbm.at[posr_v.at[c]], sem2)
        if c + 1 < N_GCH:
            if c >= 1:
                hs[c - 1].wait()
            hg[c + 1] = pltpu.async_copy(
                x_hbm.at[tokr_v.at[c + 1]], bufs[(c + 1) % 2], sem)
    hs[N_GCH - 2].wait()
    hs[N_GCH - 1].wait()

    @pl.when(wid == 0)
    def _():
        def expert_step(e, carry):
            cur, last = carry
            ev = jnp.full((16,), e, jnp.int32)
            nb = plsc.load_gather(nblk_v, [ev])[0]

            def blk_step(k, _):
                plsc.store_scatter(
                    be_v, [jnp.full((16,), cur + k, jnp.int32)], ev,
                    mask=lane0)
                return 0

            lax.fori_loop(0, nb, blk_step, 0)
            return cur + nb, jnp.where(nb > 0, e, last)

        cur, last = lax.fori_loop(0, N_EXP, expert_step, (0, 0))
        lastv = jnp.full((16,), last, jnp.int32)

        def fill_step(b, _):
            plsc.store_scatter(
                be_v, [jnp.full((16,), b, jnp.int32)], lastv, mask=lane0)
            return 0

        lax.fori_loop(cur, NB, fill_step, 0)

        curv = jnp.full((16,), cur, jnp.int32)
        for j in range(NB // 16):
            bv = jnp.full((16,), 16 * j, jnp.int32) + lane
            br_v[pl.ds(16 * j, 16)] = jnp.where(
                bv < curv, bv, jnp.full((16,), NB, jnp.int32))
        pltpu.sync_copy(be_v, be_hbm)
        pltpu.sync_copy(br_v, br_hbm)


@functools.cache
def _make_route():
    mesh = plsc.VectorSubcoreMesh(
        core_axis_name="c", subcore_axis_name="s",
        num_cores=1, num_subcores=A_TILES)
    return functools.partial(
        pl.kernel,
        mesh=mesh,
        out_type=(
            jax.ShapeDtypeStruct((NP_ROWS, D_MODEL), jnp.float32),
            jax.ShapeDtypeStruct((NA,), jnp.int32),
            jax.ShapeDtypeStruct((NB,), jnp.int32),
            jax.ShapeDtypeStruct((NB,), jnp.int32),
        ),
        scratch_types=[
            pltpu.VMEM((A_PER,), jnp.int32),
            pltpu.VMEM((A_PER,), jnp.int32),
            pltpu.VMEM((N_EXP,), jnp.int32),
            pltpu.VMEM_SHARED((A_TILES * N_EXP,), jnp.int32),
            pltpu.VMEM((A_TILES * N_EXP,), jnp.int32),
            pltpu.VMEM((N_EXP,), jnp.int32),
            pltpu.VMEM((N_EXP,), jnp.int32),
            pltpu.VMEM((A_PER,), jnp.int32),
            pltpu.VMEM((N_GCH, GCH), jnp.int32),
            pltpu.VMEM((N_GCH, GCH), jnp.int32),
            pltpu.VMEM((GCH, D_MODEL), jnp.float32),
            pltpu.VMEM((GCH, D_MODEL), jnp.float32),
            pltpu.VMEM((NB,), jnp.int32),
            pltpu.VMEM((NB,), jnp.int32),
            pltpu.SemaphoreType.DMA,
            pltpu.SemaphoreType.DMA,
        ],
        compiler_params=pltpu.CompilerParams(needs_layout_passes=False),
    )(_route_body)


def _ffn_body(be_ref, br_ref, xs_ref, w1_ref, w2_ref, w3_ref, out_ref):
    del be_ref, br_ref
    xb = xs_ref[0]
    g = jax.nn.silu(jnp.dot(xb, w1_ref[0], preferred_element_type=jnp.float32))
    v = jnp.dot(xb, w2_ref[0], preferred_element_type=jnp.float32)
    out_ref[0] = jnp.dot(g * v, w3_ref[0], preferred_element_type=jnp.float32)


@jax.jit
def _grouped_ffn(block_expert, block_row, xs, w1, w2, w3):
    grid_spec = pltpu.PrefetchScalarGridSpec(
        num_scalar_prefetch=2,
        grid=(NB,),
        in_specs=[
            pl.BlockSpec((1, B_ROWS, D_MODEL), lambda b, be, br: (br[b], 0, 0)),
            pl.BlockSpec((1, D_MODEL, D_FF), lambda b, be, br: (be[b], 0, 0)),
            pl.BlockSpec((1, D_MODEL, D_FF), lambda b, be, br: (be[b], 0, 0)),
            pl.BlockSpec((1, D_FF, D_MODEL), lambda b, be, br: (be[b], 0, 0)),
        ],
        out_specs=pl.BlockSpec(
            (1, B_ROWS, D_MODEL), lambda b, be, br: (br[b], 0, 0)),
    )
    return pl.pallas_call(
        _ffn_body,
        grid_spec=grid_spec,
        out_shape=jax.ShapeDtypeStruct((NB + 1, B_ROWS, D_MODEL), jnp.float32),
        compiler_params=pltpu.CompilerParams(
            dimension_semantics=("arbitrary",)),
    )(block_expert, block_row, xs, w1, w2, w3)


C_TILES = 32
C_PER = NA // C_TILES
C_GCH = 32


C_TOK = NT // C_TILES
C_TCH = 16
N_CCH = C_TOK // C_TCH


def _ungroup_body(ys_hbm, pos_hbm, wt_hbm, out_hbm, pos_v, wa_v, wb_v,
                  rowsa_v, rowsb_v, rowsa2_v, rowsb2_v, outc_v, outc2_v,
                  sem, sem2, sem3):
    wid = lax.axis_index("s") * 2 + lax.axis_index("c")
    t0 = wid * C_TOK
    pltpu.sync_copy(pos_hbm.at[pl.ds(t0, C_TOK)], pos_v.at[pl.ds(0, C_TOK)])
    pltpu.sync_copy(pos_hbm.at[pl.ds(NT + t0, C_TOK)],
                    pos_v.at[pl.ds(C_TOK, C_TOK)])
    pltpu.sync_copy(wt_hbm.at[0, pl.ds(t0, C_TOK)], wa_v)
    pltpu.sync_copy(wt_hbm.at[1, pl.ds(t0, C_TOK)], wb_v)

    bufs = ((rowsa_v, rowsb_v), (rowsa2_v, rowsb2_v))

    def start_gather(c, buf):
        ha = pltpu.async_copy(
            ys_hbm.at[pos_v.at[pl.ds(c * C_TCH, C_TCH)]], buf[0], sem)
        hb = pltpu.async_copy(
            ys_hbm.at[pos_v.at[pl.ds(C_TOK + c * C_TCH, C_TCH)]], buf[1],
            sem2)
        return ha, hb

    outbufs = (outc_v, outc2_v)
    hg = [None] * N_CCH
    hs = [None] * N_CCH
    hg[0] = start_gather(0, bufs[0])
    for c in range(N_CCH):
        ra, rb = bufs[c % 2]
        outc = outbufs[c % 2]
        hg[c][0].wait()
        hg[c][1].wait()
        if c >= 2:
            hs[c - 2].wait()
        zf = jnp.zeros((16,), jnp.float32)
        wcha = wa_v[pl.ds(c * C_TCH, C_TCH)]
        wchb = wb_v[pl.ds(c * C_TCH, C_TCH)]
        was = [zf + wcha[r] for r in range(C_TCH)]
        wbs = [zf + wchb[r] for r in range(C_TCH)]

        def qstep(q, carry):
            for r in range(C_TCH):
                a = ra[r, pl.ds(16 * q, 16)]
                b = rb[r, pl.ds(16 * q, 16)]
                outc[r, pl.ds(16 * q, 16)] = a * was[r] + b * wbs[r]
            return carry

        lax.fori_loop(0, D_MODEL // 16, qstep, 0)
        hs[c] = pltpu.async_copy(
            outc, out_hbm.at[pl.ds(t0 + c * C_TCH, C_TCH)], sem3)
        if c + 1 < N_CCH:
            hg[c + 1] = start_gather(c + 1, bufs[(c + 1) % 2])
    hs[N_CCH - 2].wait()
    hs[N_CCH - 1].wait()


@functools.cache
def _make_ungroup():
    mesh = plsc.VectorSubcoreMesh(
        core_axis_name="c", subcore_axis_name="s",
        num_cores=2, num_subcores=16)
    return functools.partial(
        pl.kernel,
        mesh=mesh,
        out_type=jax.ShapeDtypeStruct((NT, D_MODEL), jnp.float32),
        scratch_types=[
            pltpu.VMEM((C_PER,), jnp.int32),
            pltpu.VMEM((C_TOK,), jnp.float32),
            pltpu.VMEM((C_TOK,), jnp.float32),
            pltpu.VMEM((C_TCH, D_MODEL), jnp.float32),
            pltpu.VMEM((C_TCH, D_MODEL), jnp.float32),
            pltpu.VMEM((C_TCH, D_MODEL), jnp.float32),
            pltpu.VMEM((C_TCH, D_MODEL), jnp.float32),
            pltpu.VMEM((C_TCH, D_MODEL), jnp.float32),
            pltpu.VMEM((C_TCH, D_MODEL), jnp.float32),
            pltpu.SemaphoreType.DMA,
            pltpu.SemaphoreType.DMA,
            pltpu.SemaphoreType.DMA,
        ],
        compiler_params=pltpu.CompilerParams(needs_layout_passes=False),
    )(_ungroup_body)


def kernel(x, expert_indices, expert_weights, w1, w2, w3):
    n_tokens, d_model = x.shape
    ei_flat = expert_indices.T.reshape(-1).astype(jnp.int32)

    xs, pos, block_expert, block_row = _make_route()(ei_flat, x)
    xs = xs.reshape(NB + 1, B_ROWS, d_model)

    ys = _grouped_ffn(block_expert, block_row, xs, w1, w2, w3)
    ys = ys.reshape(NP_ROWS, d_model)

    return _make_ungroup()(ys, pos, expert_weights.T)
